# baseline (speedup 1.0000x reference)
import functools

import jax
import jax.numpy as jnp
from jax import lax
from jax.experimental import pallas as pl
from jax.experimental.pallas import tpu as pltpu
from jax.experimental.pallas import tpu_sc as plsc

PAD_INDEX = 0
BATCH, SEQ = 4096, 200
NUM_EMB, DIM = 256, 128
NC, NS, L = 2, 16, 16
NW = NC * NS
RPW = BATCH // NW


def _mask(x):
    return lax.shift_right_logical(x | (0 - x), 31).astype(jnp.int32)


def _pos_chunks(inp_v, r, pos_v):
    carry = jnp.int32(0)
    for j in range(11):
        x = inp_v[r, pl.ds(j * L, L)]
        m = _mask(x)
        c = plsc.cumsum(m) + carry
        pos_v[j // 8, pl.ds((j % 8) * L, L)] = c * m
        carry = carry + jnp.sum(m)
    x = inp_v[r, pl.ds(176, L)]
    m = _mask(x)
    c = plsc.cumsum(m) + carry
    pos_v[1, pl.ds(48, L)] = c * m
    lane = lax.iota(jnp.int32, L)
    first8 = lax.shift_right_logical(lane - 8, 31).astype(jnp.int32)
    carry_t = carry + jnp.sum(m * first8)
    x = inp_v[r, pl.ds(184, L)]
    m = _mask(x)
    c = plsc.cumsum(m) + carry_t
    pos_v[1, pl.ds(56, L)] = c * m


def kernel(input, table):
    mesh = plsc.VectorSubcoreMesh(core_axis_name="c", subcore_axis_name="s")

    @functools.partial(
        pl.kernel,
        out_type=jax.ShapeDtypeStruct((BATCH, SEQ, DIM), jnp.float32),
        mesh=mesh,
        compiler_params=pltpu.CompilerParams(needs_layout_passes=False),
        scratch_types=[
            pltpu.VMEM((RPW, SEQ), jnp.int32),
            pltpu.VMEM((2, 128), jnp.int32),
            pltpu.VMEM((2, 128), jnp.int32),
            pltpu.VMEM((SEQ, DIM), jnp.float32),
            pltpu.VMEM((SEQ, DIM), jnp.float32),
            pltpu.SemaphoreType.DMA,
            pltpu.SemaphoreType.DMA,
            pltpu.SemaphoreType.DMA,
            pltpu.SemaphoreType.DMA,
        ],
    )
    def run(inp_hbm, tbl_hbm, out_hbm, inp_v, pos0, pos1, rows0, rows1,
            sg0, sg1, ss0, ss1):
        wid = lax.axis_index("s") * NC + lax.axis_index("c")
        base = wid * RPW
        pltpu.sync_copy(inp_hbm.at[pl.ds(base, RPW)], inp_v)

        def gather_start(pos_v, rows_v, sem):
            pltpu.async_copy(
                tbl_hbm.at[pos_v.at[0]], rows_v.at[pl.ds(0, 128)], sem)
            pltpu.async_copy(
                tbl_hbm.at[pos_v.at[1, pl.ds(0, 72)]],
                rows_v.at[pl.ds(128, 72)], sem)

        def gather_wait(pos_v, rows_v, sem):
            pltpu.make_async_copy(
                tbl_hbm.at[pos_v.at[0]], rows_v.at[pl.ds(0, 128)], sem).wait()
            pltpu.make_async_copy(
                tbl_hbm.at[pos_v.at[1, pl.ds(0, 72)]],
                rows_v.at[pl.ds(128, 72)], sem).wait()

        def store_start(rows_v, r, sem):
            pltpu.async_copy(rows_v, out_hbm.at[base + r], sem)

        def store_wait(rows_v, r, sem):
            pltpu.make_async_copy(rows_v, out_hbm.at[base + r], sem).wait()

        _pos_chunks(inp_v, 0, pos0)
        gather_start(pos0, rows0, sg0)
        gather_wait(pos0, rows0, sg0)
        store_start(rows0, 0, ss0)
        _pos_chunks(inp_v, 1, pos1)
        gather_start(pos1, rows1, sg1)

        bufs = ((pos1, rows1, sg1, ss1), (pos0, rows0, sg0, ss0))

        def phase(r, a, b):
            pos_a, rows_a, sg_a, ss_a = a
            pos_b, rows_b, sg_b, ss_b = b
            gather_wait(pos_a, rows_a, sg_a)
            store_start(rows_a, r, ss_a)
            _pos_chunks(inp_v, r + 1, pos_b)
            store_wait(rows_b, r - 1, ss_b)
            gather_start(pos_b, rows_b, sg_b)

        def pair_body(i, _):
            r = 1 + 2 * i
            phase(r, bufs[0], bufs[1])
            phase(r + 1, bufs[1], bufs[0])
            return 0

        lax.fori_loop(0, (RPW - 2) // 2, pair_body, 0)

        last = RPW - 1
        gather_wait(pos1, rows1, sg1)
        store_start(rows1, last, ss1)
        store_wait(rows0, last - 1, ss0)
        store_wait(rows1, last, ss1)

    return run(input, table)


# speedup vs baseline: 4.6213x; 4.6213x over previous
import functools

import jax
import jax.numpy as jnp
from jax import lax
from jax.experimental import pallas as pl
from jax.experimental.pallas import tpu as pltpu
from jax.experimental.pallas import tpu_sc as plsc

PAD_INDEX = 0
BATCH, SEQ = 4096, 200
NUM_EMB, DIM = 256, 128
NC, NS, L = 2, 16, 16
NW = NC * NS
RPW = BATCH // NW


def _mask(x):
    return lax.shift_right_logical(x | (0 - x), 31).astype(jnp.int32)


def _pos_chunks(inp_v, r, pos_v):
    carry = jnp.int32(0)
    for j in range(11):
        x = inp_v[r, pl.ds(j * L, L)]
        m = _mask(x)
        c = plsc.cumsum(m) + carry
        pos_v[j // 8, pl.ds((j % 8) * L, L)] = c * m
        carry = carry + jnp.sum(m)
    x = inp_v[r, pl.ds(176, L)]
    m = _mask(x)
    c = plsc.cumsum(m) + carry
    pos_v[1, pl.ds(48, L)] = c * m
    lane = lax.iota(jnp.int32, L)
    first8 = lax.shift_right_logical(lane - 8, 31).astype(jnp.int32)
    carry_t = carry + jnp.sum(m * first8)
    x = inp_v[r, pl.ds(184, L)]
    m = _mask(x)
    c = plsc.cumsum(m) + carry_t
    pos_v[1, pl.ds(56, L)] = c * m


def kernel(input, table):
    mesh = plsc.VectorSubcoreMesh(core_axis_name="c", subcore_axis_name="s")

    @functools.partial(
        pl.kernel,
        out_type=jax.ShapeDtypeStruct((BATCH, SEQ, DIM), jnp.float32),
        mesh=mesh,
        compiler_params=pltpu.CompilerParams(needs_layout_passes=False),
        scratch_types=[
            pltpu.VMEM_SHARED((NUM_EMB, DIM), jnp.float32),
            pltpu.VMEM((RPW, SEQ), jnp.int32),
            pltpu.VMEM((2, 128), jnp.int32),
            pltpu.VMEM((2, 128), jnp.int32),
            pltpu.VMEM((SEQ, DIM), jnp.float32),
            pltpu.VMEM((SEQ, DIM), jnp.float32),
            pltpu.SemaphoreType.DMA,
            pltpu.SemaphoreType.DMA,
            pltpu.SemaphoreType.DMA,
            pltpu.SemaphoreType.DMA,
        ],
    )
    def run(inp_hbm, tbl_hbm, out_hbm, tbl_s, inp_v, pos0, pos1, rows0, rows1,
            sg0, sg1, ss0, ss1):
        sid = lax.axis_index("s")
        wid = sid * NC + lax.axis_index("c")
        base = wid * RPW

        @pl.when(sid == 0)
        def _stage_table():
            pltpu.sync_copy(tbl_hbm, tbl_s)

        pltpu.sync_copy(inp_hbm.at[pl.ds(base, RPW)], inp_v)
        plsc.subcore_barrier()

        def gather_start(pos_v, rows_v, sem):
            pltpu.async_copy(
                tbl_s.at[pos_v.at[0]], rows_v.at[pl.ds(0, 128)], sem)
            pltpu.async_copy(
                tbl_s.at[pos_v.at[1, pl.ds(0, 72)]],
                rows_v.at[pl.ds(128, 72)], sem)

        def gather_wait(pos_v, rows_v, sem):
            pltpu.make_async_copy(
                tbl_s.at[pos_v.at[0]], rows_v.at[pl.ds(0, 128)], sem).wait()
            pltpu.make_async_copy(
                tbl_s.at[pos_v.at[1, pl.ds(0, 72)]],
                rows_v.at[pl.ds(128, 72)], sem).wait()

        def store_start(rows_v, r, sem):
            pltpu.async_copy(rows_v, out_hbm.at[base + r], sem)

        def store_wait(rows_v, r, sem):
            pltpu.make_async_copy(rows_v, out_hbm.at[base + r], sem).wait()

        _pos_chunks(inp_v, 0, pos0)
        gather_start(pos0, rows0, sg0)
        gather_wait(pos0, rows0, sg0)
        store_start(rows0, 0, ss0)
        _pos_chunks(inp_v, 1, pos1)
        gather_start(pos1, rows1, sg1)

        bufs = ((pos1, rows1, sg1, ss1), (pos0, rows0, sg0, ss0))

        def phase(r, a, b):
            pos_a, rows_a, sg_a, ss_a = a
            pos_b, rows_b, sg_b, ss_b = b
            gather_wait(pos_a, rows_a, sg_a)
            store_start(rows_a, r, ss_a)
            _pos_chunks(inp_v, r + 1, pos_b)
            store_wait(rows_b, r - 1, ss_b)
            gather_start(pos_b, rows_b, sg_b)

        def pair_body(i, _):
            r = 1 + 2 * i
            phase(r, bufs[0], bufs[1])
            phase(r + 1, bufs[1], bufs[0])
            return 0

        lax.fori_loop(0, (RPW - 2) // 2, pair_body, 0)

        last = RPW - 1
        gather_wait(pos1, rows1, sg1)
        store_start(rows1, last, ss1)
        store_wait(rows0, last - 1, ss0)
        store_wait(rows1, last, ss1)

    return run(input, table)


# speedup vs baseline: 5.7278x; 1.2394x over previous
import functools

import jax
import jax.numpy as jnp
from jax import lax
from jax.experimental import pallas as pl
from jax.experimental.pallas import tpu as pltpu
from jax.experimental.pallas import tpu_sc as plsc

PAD_INDEX = 0
BATCH, SEQ = 4096, 200
NUM_EMB, DIM = 256, 128
NC, NS, L = 2, 16, 16
NW = NC * NS
RPW = BATCH // NW
KWIN = 8


def _mask(x):
    return lax.shift_right_logical(x | (0 - x), 31).astype(jnp.int32)


def _row_total(inp_v, r):
    t = jnp.int32(0)
    for j in range(12):
        t = t + jnp.sum(_mask(inp_v[r, pl.ds(j * L, L)]))
    m = _mask(inp_v[r, pl.ds(184, L)])
    lane = lax.iota(jnp.int32, L)
    last8 = lax.shift_right_logical(7 - lane, 31).astype(jnp.int32)
    return t + jnp.sum(m * last8)


def _pos_chunks(inp_v, r, pos_v):
    carry = jnp.int32(0)
    for j in range(11):
        x = inp_v[r, pl.ds(j * L, L)]
        m = _mask(x)
        c = plsc.cumsum(m) + carry
        pos_v[j // 8, pl.ds((j % 8) * L, L)] = c * m
        carry = carry + jnp.sum(m)
    x = inp_v[r, pl.ds(176, L)]
    m = _mask(x)
    c = plsc.cumsum(m) + carry
    pos_v[1, pl.ds(48, L)] = c * m
    lane = lax.iota(jnp.int32, L)
    first8 = lax.shift_right_logical(lane - 8, 31).astype(jnp.int32)
    carry_t = carry + jnp.sum(m * first8)
    x = inp_v[r, pl.ds(184, L)]
    m = _mask(x)
    c = plsc.cumsum(m) + carry_t
    pos_v[1, pl.ds(56, L)] = c * m


def kernel(input, table):
    mesh = plsc.VectorSubcoreMesh(core_axis_name="c", subcore_axis_name="s")

    @functools.partial(
        pl.kernel,
        out_type=jax.ShapeDtypeStruct((BATCH, SEQ, DIM), jnp.float32),
        mesh=mesh,
        compiler_params=pltpu.CompilerParams(needs_layout_passes=False),
        scratch_types=[
            pltpu.VMEM_SHARED((NUM_EMB, DIM), jnp.float32),
            pltpu.VMEM((RPW, SEQ), jnp.int32),
            pltpu.VMEM((SEQ, DIM), jnp.float32),
            pltpu.VMEM((2, 128), jnp.int32),
            pltpu.VMEM((SEQ, DIM), jnp.float32),
            pltpu.SemaphoreType.DMA,
            pltpu.SemaphoreType.DMA,
            pltpu.SemaphoreType.DMA,
        ],
    )
    def run(inp_hbm, tbl_hbm, out_hbm, tbl_s, inp_v, tbl_fast, pos_v, rows_v,
            sf, sg, ss):
        sid = lax.axis_index("s")
        wid = sid * NC + lax.axis_index("c")
        base = wid * RPW

        @pl.when(sid == 0)
        def _stage_table():
            pltpu.sync_copy(tbl_hbm, tbl_s)

        pltpu.sync_copy(inp_hbm.at[pl.ds(base, RPW)], inp_v)
        plsc.subcore_barrier()

        def gather_start(pos_ref, rows_ref, sem):
            pltpu.async_copy(
                tbl_s.at[pos_ref.at[0]], rows_ref.at[pl.ds(0, 128)], sem)
            pltpu.async_copy(
                tbl_s.at[pos_ref.at[1, pl.ds(0, 72)]],
                rows_ref.at[pl.ds(128, 72)], sem)

        def gather_wait(pos_ref, rows_ref, sem):
            pltpu.make_async_copy(
                tbl_s.at[pos_ref.at[0]], rows_ref.at[pl.ds(0, 128)],
                sem).wait()
            pltpu.make_async_copy(
                tbl_s.at[pos_ref.at[1, pl.ds(0, 72)]],
                rows_ref.at[pl.ds(128, 72)], sem).wait()

        lane = lax.iota(jnp.int32, L)
        for j in range(13):
            pos_v[j // 8, pl.ds((j % 8) * L, L)] = lane + (j * L + 1)
        gather_start(pos_v, tbl_fast, sg)
        gather_wait(pos_v, tbl_fast, sg)

        def row_body(r, carry):
            nout, pending = carry
            total = _row_total(inp_v, r)
            fast = total == SEQ

            @pl.when(fast)
            def _fast():
                pltpu.async_copy(tbl_fast, out_hbm.at[base + r], sf)

            @pl.when(fast & (nout >= KWIN))
            def _drain_one():
                pltpu.make_async_copy(tbl_fast, out_hbm.at[base + r],
                                      sf).wait()

            @pl.when(jnp.logical_not(fast))
            def _slow():
                @pl.when(pending == 1)
                def _wait_prev():
                    pltpu.make_async_copy(rows_v, out_hbm.at[base + r],
                                          ss).wait()

                _pos_chunks(inp_v, r, pos_v)
                gather_start(pos_v, rows_v, sg)
                gather_wait(pos_v, rows_v, sg)
                pltpu.async_copy(rows_v, out_hbm.at[base + r], ss)

            nout = jnp.where(fast, jnp.minimum(nout + 1, KWIN), nout)
            pending = jnp.where(fast, pending, jnp.int32(1))
            return nout, pending

        nout, pending = lax.fori_loop(
            0, RPW, row_body, (jnp.int32(0), jnp.int32(0)))

        @pl.when(pending == 1)
        def _final_slow():
            pltpu.make_async_copy(rows_v, out_hbm.at[base], ss).wait()

        def drain_body(i, _):
            pltpu.make_async_copy(tbl_fast, out_hbm.at[base], sf).wait()
            return 0

        lax.fori_loop(0, nout, drain_body, 0)

    return run(input, table)


# speedup vs baseline: 5.8116x; 1.0146x over previous
import functools

import jax
import jax.numpy as jnp
from jax import lax
from jax.experimental import pallas as pl
from jax.experimental.pallas import tpu as pltpu
from jax.experimental.pallas import tpu_sc as plsc

PAD_INDEX = 0
BATCH, SEQ = 4096, 200
NUM_EMB, DIM = 256, 128
NC, NS, L = 2, 16, 16
NW = NC * NS
RPW = BATCH // NW
KWIN = 12


def _mask(x):
    return lax.shift_right_logical(x | (0 - x), 31).astype(jnp.int32)


def _row_total(inp_v, r):
    t = jnp.int32(0)
    for j in range(12):
        t = t + jnp.sum(_mask(inp_v[r, pl.ds(j * L, L)]))
    m = _mask(inp_v[r, pl.ds(184, L)])
    lane = lax.iota(jnp.int32, L)
    last8 = lax.shift_right_logical(7 - lane, 31).astype(jnp.int32)
    return t + jnp.sum(m * last8)


def _pos_chunks(inp_v, r, pos_v):
    carry = jnp.int32(0)
    for j in range(11):
        x = inp_v[r, pl.ds(j * L, L)]
        m = _mask(x)
        c = plsc.cumsum(m) + carry
        pos_v[j // 8, pl.ds((j % 8) * L, L)] = c * m
        carry = carry + jnp.sum(m)
    x = inp_v[r, pl.ds(176, L)]
    m = _mask(x)
    c = plsc.cumsum(m) + carry
    pos_v[1, pl.ds(48, L)] = c * m
    lane = lax.iota(jnp.int32, L)
    first8 = lax.shift_right_logical(lane - 8, 31).astype(jnp.int32)
    carry_t = carry + jnp.sum(m * first8)
    x = inp_v[r, pl.ds(184, L)]
    m = _mask(x)
    c = plsc.cumsum(m) + carry_t
    pos_v[1, pl.ds(56, L)] = c * m


def kernel(input, table):
    mesh = plsc.VectorSubcoreMesh(core_axis_name="c", subcore_axis_name="s")

    @functools.partial(
        pl.kernel,
        out_type=jax.ShapeDtypeStruct((BATCH, SEQ, DIM), jnp.float32),
        mesh=mesh,
        compiler_params=pltpu.CompilerParams(needs_layout_passes=False),
        scratch_types=[
            pltpu.VMEM_SHARED((NUM_EMB, DIM), jnp.float32),
            pltpu.VMEM((RPW, SEQ), jnp.int32),
            pltpu.VMEM((SEQ, DIM), jnp.float32),
            pltpu.VMEM((2, 128), jnp.int32),
            pltpu.VMEM((SEQ, DIM), jnp.float32),
            pltpu.SemaphoreType.DMA,
            pltpu.SemaphoreType.DMA,
            pltpu.SemaphoreType.DMA,
        ],
    )
    def run(inp_hbm, tbl_hbm, out_hbm, tbl_s, inp_v, tbl_fast, pos_v, rows_v,
            sf, sg, ss):
        sid = lax.axis_index("s")
        wid = sid * NC + lax.axis_index("c")
        base = wid * RPW

        @pl.when(sid == 0)
        def _stage_table():
            pltpu.sync_copy(tbl_hbm, tbl_s)

        pltpu.sync_copy(inp_hbm.at[pl.ds(base, RPW)], inp_v)
        plsc.subcore_barrier()

        def gather_start(pos_ref, rows_ref, sem):
            pltpu.async_copy(
                tbl_s.at[pos_ref.at[0]], rows_ref.at[pl.ds(0, 128)], sem)
            pltpu.async_copy(
                tbl_s.at[pos_ref.at[1, pl.ds(0, 72)]],
                rows_ref.at[pl.ds(128, 72)], sem)

        def gather_wait(pos_ref, rows_ref, sem):
            pltpu.make_async_copy(
                tbl_s.at[pos_ref.at[0]], rows_ref.at[pl.ds(0, 128)],
                sem).wait()
            pltpu.make_async_copy(
                tbl_s.at[pos_ref.at[1, pl.ds(0, 72)]],
                rows_ref.at[pl.ds(128, 72)], sem).wait()

        lane = lax.iota(jnp.int32, L)
        for j in range(13):
            pos_v[j // 8, pl.ds((j % 8) * L, L)] = lane + (j * L + 1)
        gather_start(pos_v, tbl_fast, sg)
        gather_wait(pos_v, tbl_fast, sg)

        def row_body(r, carry):
            nout, pending, gpend, rslow = carry

            @pl.when(gpend == 1)
            def _resolve():
                gather_wait(pos_v, rows_v, sg)
                pltpu.async_copy(rows_v, out_hbm.at[base + rslow], ss)

            pending = jnp.maximum(pending, gpend)
            total = _row_total(inp_v, r)
            fast = total == SEQ

            @pl.when(fast)
            def _fast():
                pltpu.async_copy(tbl_fast, out_hbm.at[base + r], sf)

            @pl.when(fast & (nout >= KWIN))
            def _drain_one():
                pltpu.make_async_copy(tbl_fast, out_hbm.at[base + r],
                                      sf).wait()

            @pl.when(jnp.logical_not(fast))
            def _slow():
                @pl.when(pending == 1)
                def _wait_prev():
                    pltpu.make_async_copy(rows_v, out_hbm.at[base + r],
                                          ss).wait()

                _pos_chunks(inp_v, r, pos_v)
                gather_start(pos_v, rows_v, sg)

            nout = jnp.where(fast, jnp.minimum(nout + 1, KWIN), nout)
            pending = jnp.where(fast, pending, jnp.int32(0))
            gpend = jnp.where(fast, jnp.int32(0), jnp.int32(1))
            rslow = jnp.where(fast, rslow, r)
            return nout, pending, gpend, rslow

        nout, pending, gpend, rslow = lax.fori_loop(
            0, RPW, row_body,
            (jnp.int32(0), jnp.int32(0), jnp.int32(0), jnp.int32(0)))

        @pl.when(gpend == 1)
        def _final_gather():
            gather_wait(pos_v, rows_v, sg)
            pltpu.async_copy(rows_v, out_hbm.at[base + rslow], ss)
            pltpu.make_async_copy(rows_v, out_hbm.at[base + rslow],
                                  ss).wait()

        @pl.when((gpend == 0) & (pending == 1))
        def _final_slow():
            pltpu.make_async_copy(rows_v, out_hbm.at[base], ss).wait()

        def drain_body(i, _):
            pltpu.make_async_copy(tbl_fast, out_hbm.at[base], sf).wait()
            return 0

        lax.fori_loop(0, nout, drain_body, 0)

    return run(input, table)
